# baseline (speedup 1.0000x reference)
import functools

import jax
import jax.numpy as jnp
from jax import lax
from jax.experimental import pallas as pl
from jax.experimental.pallas import tpu as pltpu
from jax.experimental.pallas import tpu_sc as plsc

N = 10000
E = 320000
FIN = 128
HID = 128
OUT = 64

NC = 2
NS = 16
NW = NC * NS
CHUNK = 128
EDGES_PER_TILE = E // NW
NCHUNK = 80
EP_TILE = NCHUNK * CHUNK
EP = EP_TILE * NW
NPAD = 10112
ROWS_PER_TILE = NPAD // NS


def _sc_mesh():
    return plsc.VectorSubcoreMesh(
        core_axis_name="c", subcore_axis_name="s", num_cores=NC, num_subcores=NS)


def _deg_body(cx_hbm, cy_hbm, ones_hbm, zeros_hbm, out_hbm, idx_v, ones_v, acc, sem):
    c = lax.axis_index("c")
    s = lax.axis_index("s")
    wid = c * NS + s

    @pl.when(s == 0)
    def _zero():
        pltpu.sync_copy(zeros_hbm, acc)

    pltpu.sync_copy(ones_hbm, ones_v)
    plsc.subcore_barrier()
    for col_hbm in (cx_hbm, cy_hbm):
        pltpu.sync_copy(col_hbm.at[wid], idx_v)

        def body(j, carry):
            pltpu.sync_copy(ones_v, acc.at[idx_v.at[j]], add=True)
            return carry

        lax.fori_loop(0, NCHUNK, body, 0)
    plsc.subcore_barrier()

    @pl.when(s == 0)
    def _out():
        pltpu.sync_copy(acc, out_hbm.at[c, 0])


def _sc_degrees(cx, cy, ones, zeros):
    return pl.kernel(
        _deg_body,
        out_type=jax.ShapeDtypeStruct((NC, 1, 2 * NPAD), jnp.float32),
        mesh=_sc_mesh(),
        scratch_types=[
            pltpu.VMEM((NCHUNK, CHUNK), jnp.int32),
            pltpu.VMEM((CHUNK,), jnp.float32),
            pltpu.VMEM_SHARED((2 * NPAD,), jnp.float32),
            pltpu.SemaphoreType.DMA,
        ],
    )(cx, cy, ones, zeros)


GSIZE = 8
NGROUP = NCHUNK // GSIZE


def _scatter_body(d, u_hbm, row_hbm, col_hbm, zeros_hbm, out_hbm,
                  rgrp, cidx, buf, acc, sem):
    c = lax.axis_index("c")
    s = lax.axis_index("s")
    wid = c * NS + s
    pltpu.sync_copy(zeros_hbm.at[pl.ds(s * ROWS_PER_TILE, ROWS_PER_TILE)],
                    acc.at[pl.ds(s * ROWS_PER_TILE, ROWS_PER_TILE)])
    pltpu.sync_copy(col_hbm.at[wid], cidx)
    plsc.subcore_barrier()

    def group(g, carry):
        pltpu.sync_copy(row_hbm.at[wid, pl.ds(g * GSIZE, GSIZE)], rgrp)
        for k in range(GSIZE):
            jj = g * GSIZE + k
            pltpu.async_copy(u_hbm.at[rgrp.at[k]], buf, sem).wait()
            pltpu.sync_copy(buf, acc.at[cidx.at[jj]], add=True)
        return carry

    lax.fori_loop(0, NGROUP, group, 0)
    plsc.subcore_barrier()
    pltpu.sync_copy(acc.at[pl.ds(s * ROWS_PER_TILE, ROWS_PER_TILE)],
                    out_hbm.at[c, pl.ds(s * ROWS_PER_TILE, ROWS_PER_TILE)])


def _sc_scatter(u, rowi, coli, zeros, d):
    return pl.kernel(
        functools.partial(_scatter_body, d),
        out_type=jax.ShapeDtypeStruct((NC, NPAD, d), jnp.float32),
        mesh=_sc_mesh(),
        scratch_types=[
            pltpu.VMEM((GSIZE, CHUNK), jnp.int32),
            pltpu.VMEM((NCHUNK, CHUNK), jnp.int32),
            pltpu.VMEM((CHUNK, d), jnp.float32),
            pltpu.VMEM_SHARED((NPAD, d), jnp.float32),
            pltpu.SemaphoreType.DMA,
        ],
    )(u, rowi, coli, zeros)


BLK = 1000


def _dinv_kernel(deg_ref, o_ref):
    d = deg_ref[0, :] + deg_ref[1, :] + 1.0
    o_ref[0, :] = lax.rsqrt(d)


def _tc_dinv(degs):
    return pl.pallas_call(
        _dinv_kernel,
        out_shape=jax.ShapeDtypeStruct((1, 2 * NPAD), jnp.float32),
    )(degs)


def _first_kernel(x_ref, w_ref, dv_ref, o_ref):
    h = jnp.dot(x_ref[...], w_ref[...], preferred_element_type=jnp.float32)
    o_ref[...] = h * dv_ref[...]


def _tc_first(x, W, dv):
    m = x.shape[1]
    k = W.shape[1]
    return pl.pallas_call(
        _first_kernel,
        grid=(N // BLK,),
        in_specs=[
            pl.BlockSpec((BLK, m), lambda i: (i, 0)),
            pl.BlockSpec((m, k), lambda i: (0, 0)),
            pl.BlockSpec((BLK, 1), lambda i: (i, 0)),
        ],
        out_specs=pl.BlockSpec((BLK, k), lambda i: (i, 0)),
        out_shape=jax.ShapeDtypeStruct((N, k), jnp.float32),
    )(x, W, dv)


def _mid_kernel(p0_ref, p1_ref, u_ref, dv_ref, b_ref, w_ref, o_ref):
    z = (p0_ref[...] + p1_ref[...] + u_ref[...]) * dv_ref[...] + b_ref[...]
    a = jnp.maximum(z, 0.0)
    o_ref[...] = jnp.dot(a, w_ref[...], preferred_element_type=jnp.float32) * dv_ref[...]


def _tc_mid(p0, p1, u, dv, b, W):
    m = u.shape[1]
    k = W.shape[1]
    return pl.pallas_call(
        _mid_kernel,
        grid=(N // BLK,),
        in_specs=[
            pl.BlockSpec((BLK, m), lambda i: (i, 0)),
            pl.BlockSpec((BLK, m), lambda i: (i, 0)),
            pl.BlockSpec((BLK, m), lambda i: (i, 0)),
            pl.BlockSpec((BLK, 1), lambda i: (i, 0)),
            pl.BlockSpec((1, m), lambda i: (0, 0)),
            pl.BlockSpec((m, k), lambda i: (0, 0)),
        ],
        out_specs=pl.BlockSpec((BLK, k), lambda i: (i, 0)),
        out_shape=jax.ShapeDtypeStruct((N, k), jnp.float32),
    )(p0, p1, u, dv, b, W)


def _last_kernel(p0_ref, p1_ref, u_ref, dv_ref, b_ref, o_ref):
    o_ref[...] = (p0_ref[...] + p1_ref[...] + u_ref[...]) * dv_ref[...] + b_ref[...]


def _tc_last(p0, p1, u, dv, b):
    m = u.shape[1]
    return pl.pallas_call(
        _last_kernel,
        grid=(N // BLK,),
        in_specs=[
            pl.BlockSpec((BLK, m), lambda i: (i, 0)),
            pl.BlockSpec((BLK, m), lambda i: (i, 0)),
            pl.BlockSpec((BLK, m), lambda i: (i, 0)),
            pl.BlockSpec((BLK, 1), lambda i: (i, 0)),
            pl.BlockSpec((1, m), lambda i: (0, 0)),
        ],
        out_specs=pl.BlockSpec((BLK, m), lambda i: (i, 0)),
        out_shape=jax.ShapeDtypeStruct((N, m), jnp.float32),
    )(p0, p1, u, dv, b)


def _pad_edges(edge_index):
    pad = EP - E
    row = jnp.concatenate([edge_index[0], jnp.zeros((pad,), jnp.int32)])
    col = jnp.concatenate([edge_index[1], jnp.full((pad,), N, jnp.int32)])
    return row.reshape(NW, NCHUNK, CHUNK), col.reshape(NW, NCHUNK, CHUNK)


def _branch(x, rowi, coli, dv, W1, b1, W2, b2, W3, b3, z128):
    W3p = jnp.pad(W3, ((0, 0), (0, HID - OUT)))
    b3p = jnp.pad(b3, (0, HID - OUT))
    u1 = _tc_first(x, W1, dv)
    S1 = _sc_scatter(u1, rowi, coli, z128, HID)
    u2 = _tc_mid(S1[0, :N], S1[1, :N], u1, dv, b1.reshape(1, HID), W2)
    S2 = _sc_scatter(u2, rowi, coli, z128, HID)
    u3 = _tc_mid(S2[0, :N], S2[1, :N], u2, dv, b2.reshape(1, HID), W3p)
    S3 = _sc_scatter(u3, rowi, coli, z128, HID)
    out = _tc_last(S3[0, :N], S3[1, :N], u3, dv, b3p.reshape(1, HID))
    return out[:, :OUT]


def kernel(x_data_matrix, y_data_matrix, x_edge_index, y_edge_index,
           W1x, b1x, W2x, b2x, W3x, b3x,
           W1y, b1y, W2y, b2y, W3y, b3y):
    rx, cx = _pad_edges(x_edge_index)
    ry, cy = _pad_edges(y_edge_index)
    ones = jnp.ones((CHUNK,), jnp.float32)
    z2n = jnp.zeros((2 * NPAD,), jnp.float32)
    z128 = jnp.zeros((NPAD, HID), jnp.float32)

    degs = _sc_degrees(cx, cy + NPAD, ones, z2n).reshape(NC, 2 * NPAD)
    dinv = _tc_dinv(degs)[0]
    dvx = dinv[:N].reshape(N, 1)
    dvy = dinv[NPAD:NPAD + N].reshape(N, 1)

    xo = _branch(x_data_matrix, rx, cx, dvx, W1x, b1x, W2x, b2x, W3x, b3x, z128)
    yo = _branch(y_data_matrix, ry, cy, dvy, W1y, b1y, W2y, b2y, W3y, b3y, z128)
    return (xo, yo)


# speedup vs baseline: 1.0066x; 1.0066x over previous
import functools

import jax
import jax.numpy as jnp
from jax import lax
from jax.experimental import pallas as pl
from jax.experimental.pallas import tpu as pltpu
from jax.experimental.pallas import tpu_sc as plsc

N = 10000
E = 320000
FIN = 128
HID = 128
OUT = 64

NC = 2
NS = 16
NW = NC * NS
CHUNK = 128
EDGES_PER_TILE = E // NW
NCHUNK = 80
EP_TILE = NCHUNK * CHUNK
EP = EP_TILE * NW
NPAD = 10112
ROWS_PER_TILE = NPAD // NS


def _sc_mesh():
    return plsc.VectorSubcoreMesh(
        core_axis_name="c", subcore_axis_name="s", num_cores=NC, num_subcores=NS)


def _deg_body(cx_hbm, cy_hbm, ones_hbm, zeros_hbm, out_hbm, idx_v, ones_v, acc, sem):
    c = lax.axis_index("c")
    s = lax.axis_index("s")
    wid = c * NS + s

    @pl.when(s == 0)
    def _zero():
        pltpu.sync_copy(zeros_hbm, acc)

    pltpu.sync_copy(ones_hbm, ones_v)
    plsc.subcore_barrier()
    for col_hbm in (cx_hbm, cy_hbm):
        pltpu.sync_copy(col_hbm.at[wid], idx_v)

        def body(j, carry):
            pltpu.sync_copy(ones_v, acc.at[idx_v.at[j]], add=True)
            return carry

        lax.fori_loop(0, NCHUNK, body, 0)
    plsc.subcore_barrier()

    @pl.when(s == 0)
    def _out():
        pltpu.sync_copy(acc, out_hbm.at[c, 0])


def _sc_degrees(cx, cy, ones, zeros):
    return pl.kernel(
        _deg_body,
        out_type=jax.ShapeDtypeStruct((NC, 1, 2 * NPAD), jnp.float32),
        mesh=_sc_mesh(),
        scratch_types=[
            pltpu.VMEM((NCHUNK, CHUNK), jnp.int32),
            pltpu.VMEM((CHUNK,), jnp.float32),
            pltpu.VMEM_SHARED((2 * NPAD,), jnp.float32),
            pltpu.SemaphoreType.DMA,
        ],
    )(cx, cy, ones, zeros)


def _scatter_body(d, u_hbm, row_hbm, col_hbm, zeros_hbm, out_hbm,
                  ridx, cidx, buf, acc, sem):
    c = lax.axis_index("c")
    s = lax.axis_index("s")
    wid = c * NS + s
    pltpu.sync_copy(zeros_hbm.at[pl.ds(s * ROWS_PER_TILE, ROWS_PER_TILE)],
                    acc.at[pl.ds(s * ROWS_PER_TILE, ROWS_PER_TILE)])
    pltpu.sync_copy(row_hbm.at[wid], ridx)
    pltpu.sync_copy(col_hbm.at[wid], cidx)
    plsc.subcore_barrier()

    def body(j, carry):
        pltpu.async_copy(u_hbm.at[ridx.at[j]], buf, sem).wait()
        pltpu.sync_copy(buf, acc.at[cidx.at[j]], add=True)
        return carry

    lax.fori_loop(0, NCHUNK, body, 0)
    plsc.subcore_barrier()
    pltpu.sync_copy(acc.at[pl.ds(s * ROWS_PER_TILE, ROWS_PER_TILE)],
                    out_hbm.at[c, pl.ds(s * ROWS_PER_TILE, ROWS_PER_TILE)])


def _sc_scatter(u, rowi, coli, zeros, d):
    return pl.kernel(
        functools.partial(_scatter_body, d),
        out_type=jax.ShapeDtypeStruct((NC, NPAD, d), jnp.float32),
        mesh=_sc_mesh(),
        scratch_types=[
            pltpu.VMEM((NCHUNK, CHUNK), jnp.int32),
            pltpu.VMEM((NCHUNK, CHUNK), jnp.int32),
            pltpu.VMEM((CHUNK, d), jnp.float32),
            pltpu.VMEM_SHARED((NPAD, d), jnp.float32),
            pltpu.SemaphoreType.DMA,
        ],
    )(u, rowi, coli, zeros)


BLK = 1000


def _dinv_kernel(deg_ref, o_ref):
    d = deg_ref[0, :] + deg_ref[1, :] + 1.0
    o_ref[0, :] = lax.rsqrt(d)


def _tc_dinv(degs):
    return pl.pallas_call(
        _dinv_kernel,
        out_shape=jax.ShapeDtypeStruct((1, 2 * NPAD), jnp.float32),
    )(degs)


def _first_kernel(x_ref, w_ref, dv_ref, o_ref):
    h = jnp.dot(x_ref[...], w_ref[...], preferred_element_type=jnp.float32)
    o_ref[...] = h * dv_ref[...]


def _tc_first(x, W, dv):
    m = x.shape[1]
    k = W.shape[1]
    return pl.pallas_call(
        _first_kernel,
        grid=(N // BLK,),
        in_specs=[
            pl.BlockSpec((BLK, m), lambda i: (i, 0)),
            pl.BlockSpec((m, k), lambda i: (0, 0)),
            pl.BlockSpec((BLK, 1), lambda i: (i, 0)),
        ],
        out_specs=pl.BlockSpec((BLK, k), lambda i: (i, 0)),
        out_shape=jax.ShapeDtypeStruct((N, k), jnp.float32),
    )(x, W, dv)


def _mid_kernel(p0_ref, p1_ref, u_ref, dv_ref, b_ref, w_ref, o_ref):
    z = (p0_ref[...] + p1_ref[...] + u_ref[...]) * dv_ref[...] + b_ref[...]
    a = jnp.maximum(z, 0.0)
    o_ref[...] = jnp.dot(a, w_ref[...], preferred_element_type=jnp.float32) * dv_ref[...]


def _tc_mid(p0, p1, u, dv, b, W):
    m = u.shape[1]
    k = W.shape[1]
    return pl.pallas_call(
        _mid_kernel,
        grid=(N // BLK,),
        in_specs=[
            pl.BlockSpec((BLK, m), lambda i: (i, 0)),
            pl.BlockSpec((BLK, m), lambda i: (i, 0)),
            pl.BlockSpec((BLK, m), lambda i: (i, 0)),
            pl.BlockSpec((BLK, 1), lambda i: (i, 0)),
            pl.BlockSpec((1, m), lambda i: (0, 0)),
            pl.BlockSpec((m, k), lambda i: (0, 0)),
        ],
        out_specs=pl.BlockSpec((BLK, k), lambda i: (i, 0)),
        out_shape=jax.ShapeDtypeStruct((N, k), jnp.float32),
    )(p0, p1, u, dv, b, W)


def _last_kernel(p0_ref, p1_ref, u_ref, dv_ref, b_ref, o_ref):
    o_ref[...] = (p0_ref[...] + p1_ref[...] + u_ref[...]) * dv_ref[...] + b_ref[...]


def _tc_last(p0, p1, u, dv, b):
    m = u.shape[1]
    return pl.pallas_call(
        _last_kernel,
        grid=(N // BLK,),
        in_specs=[
            pl.BlockSpec((BLK, m), lambda i: (i, 0)),
            pl.BlockSpec((BLK, m), lambda i: (i, 0)),
            pl.BlockSpec((BLK, m), lambda i: (i, 0)),
            pl.BlockSpec((BLK, 1), lambda i: (i, 0)),
            pl.BlockSpec((1, m), lambda i: (0, 0)),
        ],
        out_specs=pl.BlockSpec((BLK, m), lambda i: (i, 0)),
        out_shape=jax.ShapeDtypeStruct((N, m), jnp.float32),
    )(p0, p1, u, dv, b)


def _pad_edges(edge_index):
    pad = EP - E
    row = jnp.concatenate([edge_index[0], jnp.zeros((pad,), jnp.int32)])
    col = jnp.concatenate([edge_index[1], jnp.full((pad,), N, jnp.int32)])
    return row.reshape(NW, NCHUNK, CHUNK), col.reshape(NW, NCHUNK, CHUNK)


def _branch(x, rowi, coli, dv, W1, b1, W2, b2, W3, b3, z128):
    W3p = jnp.pad(W3, ((0, 0), (0, HID - OUT)))
    b3p = jnp.pad(b3, (0, HID - OUT))
    u1 = _tc_first(x, W1, dv)
    S1 = _sc_scatter(u1, rowi, coli, z128, HID)
    u2 = _tc_mid(S1[0, :N], S1[1, :N], u1, dv, b1.reshape(1, HID), W2)
    S2 = _sc_scatter(u2, rowi, coli, z128, HID)
    u3 = _tc_mid(S2[0, :N], S2[1, :N], u2, dv, b2.reshape(1, HID), W3p)
    S3 = _sc_scatter(u3, rowi, coli, z128, HID)
    out = _tc_last(S3[0, :N], S3[1, :N], u3, dv, b3p.reshape(1, HID))
    return out[:, :OUT]


def kernel(x_data_matrix, y_data_matrix, x_edge_index, y_edge_index,
           W1x, b1x, W2x, b2x, W3x, b3x,
           W1y, b1y, W2y, b2y, W3y, b3y):
    rx, cx = _pad_edges(x_edge_index)
    ry, cy = _pad_edges(y_edge_index)
    ones = jnp.ones((CHUNK,), jnp.float32)
    z2n = jnp.zeros((2 * NPAD,), jnp.float32)
    z128 = jnp.zeros((NPAD, HID), jnp.float32)

    degs = _sc_degrees(cx, cy + NPAD, ones, z2n).reshape(NC, 2 * NPAD)
    dinv = _tc_dinv(degs)[0]
    dvx = dinv[:N].reshape(N, 1)
    dvy = dinv[NPAD:NPAD + N].reshape(N, 1)

    xo = _branch(x_data_matrix, rx, cx, dvx, W1x, b1x, W2x, b2x, W3x, b3x, z128)
    yo = _branch(y_data_matrix, ry, cy, dvy, W1y, b1y, W2y, b2y, W3y, b3y, z128)
    return (xo, yo)


# speedup vs baseline: 2.7391x; 2.7210x over previous
import functools

import jax
import jax.numpy as jnp
from jax import lax
from jax.experimental import pallas as pl
from jax.experimental.pallas import tpu as pltpu
from jax.experimental.pallas import tpu_sc as plsc

N = 10000
E = 320000
FIN = 128
HID = 128
OUT = 64

NC = 2
NS = 16
NW = NC * NS
CHUNK = 128
EDGES_PER_TILE = E // NW
NCHUNK = 80
EP_TILE = NCHUNK * CHUNK
EP = EP_TILE * NW
NPAD = 10112
ROWS_PER_TILE = NPAD // NS


def _sc_mesh():
    return plsc.VectorSubcoreMesh(
        core_axis_name="c", subcore_axis_name="s", num_cores=NC, num_subcores=NS)


def _deg_body(cx_hbm, cy_hbm, ones_hbm, zeros_hbm, out_hbm, idx_v, ones_v, acc, sem):
    c = lax.axis_index("c")
    s = lax.axis_index("s")
    wid = c * NS + s

    @pl.when(s == 0)
    def _zero():
        pltpu.sync_copy(zeros_hbm, acc)

    pltpu.sync_copy(ones_hbm, ones_v)
    plsc.subcore_barrier()
    for col_hbm in (cx_hbm, cy_hbm):
        pltpu.sync_copy(col_hbm.at[wid], idx_v)

        def body(j, carry):
            pltpu.sync_copy(ones_v, acc.at[idx_v.at[j]], add=True)
            return carry

        lax.fori_loop(0, NCHUNK, body, 0)
    plsc.subcore_barrier()

    @pl.when(s == 0)
    def _out():
        pltpu.sync_copy(acc, out_hbm.at[c, 0])


def _sc_degrees(cx, cy, ones, zeros):
    return pl.kernel(
        _deg_body,
        out_type=jax.ShapeDtypeStruct((NC, 1, 2 * NPAD), jnp.float32),
        mesh=_sc_mesh(),
        scratch_types=[
            pltpu.VMEM((NCHUNK, CHUNK), jnp.int32),
            pltpu.VMEM((CHUNK,), jnp.float32),
            pltpu.VMEM_SHARED((2 * NPAD,), jnp.float32),
            pltpu.SemaphoreType.DMA,
        ],
    )(cx, cy, ones, zeros)


def _scatter_body(d, u_hbm, row_hbm, col_hbm, zeros_hbm, out_hbm,
                  ridx, cidx, buf, acc, sem):
    c = lax.axis_index("c")
    s = lax.axis_index("s")
    wid = c * NS + s
    pltpu.sync_copy(zeros_hbm.at[pl.ds(s * ROWS_PER_TILE, ROWS_PER_TILE)],
                    acc.at[pl.ds(s * ROWS_PER_TILE, ROWS_PER_TILE)])
    pltpu.sync_copy(row_hbm.at[wid], ridx)
    pltpu.sync_copy(col_hbm.at[wid], cidx)
    plsc.subcore_barrier()

    def body(j, carry):
        pltpu.async_copy(u_hbm.at[ridx.at[j]], buf, sem).wait()
        pltpu.sync_copy(buf, acc.at[cidx.at[j]], add=True)
        return carry

    lax.fori_loop(0, NCHUNK, body, 0)
    plsc.subcore_barrier()
    pltpu.sync_copy(acc.at[pl.ds(s * ROWS_PER_TILE, ROWS_PER_TILE)],
                    out_hbm.at[c, pl.ds(s * ROWS_PER_TILE, ROWS_PER_TILE)])


def _sc_scatter(u, rowi, coli, zeros, d):
    return pl.kernel(
        functools.partial(_scatter_body, d),
        out_type=jax.ShapeDtypeStruct((NC, NPAD, d), jnp.float32),
        mesh=_sc_mesh(),
        scratch_types=[
            pltpu.VMEM((NCHUNK, CHUNK), jnp.int32),
            pltpu.VMEM((NCHUNK, CHUNK), jnp.int32),
            pltpu.VMEM((CHUNK, d), jnp.float32),
            pltpu.VMEM_SHARED((NPAD, d), jnp.float32),
            pltpu.SemaphoreType.DMA,
        ],
    )(u, rowi, coli, zeros)


BLK = 1000


def _dinv_kernel(deg_ref, o_ref):
    d = deg_ref[0, :] + deg_ref[1, :] + 1.0
    o_ref[0, :] = lax.rsqrt(d)


def _tc_dinv(degs):
    return pl.pallas_call(
        _dinv_kernel,
        out_shape=jax.ShapeDtypeStruct((1, 2 * NPAD), jnp.float32),
    )(degs)


def _first_kernel(x_ref, w_ref, dv_ref, o_ref):
    h = jnp.dot(x_ref[...], w_ref[...], preferred_element_type=jnp.float32)
    o_ref[...] = h * dv_ref[...]


def _tc_first(x, W, dv):
    m = x.shape[1]
    k = W.shape[1]
    return pl.pallas_call(
        _first_kernel,
        grid=(N // BLK,),
        in_specs=[
            pl.BlockSpec((BLK, m), lambda i: (i, 0)),
            pl.BlockSpec((m, k), lambda i: (0, 0)),
            pl.BlockSpec((BLK, 1), lambda i: (i, 0)),
        ],
        out_specs=pl.BlockSpec((BLK, k), lambda i: (i, 0)),
        out_shape=jax.ShapeDtypeStruct((N, k), jnp.float32),
    )(x, W, dv)


def _mid_kernel(p0_ref, p1_ref, u_ref, dv_ref, b_ref, w_ref, o_ref):
    z = (p0_ref[...] + p1_ref[...] + u_ref[...]) * dv_ref[...] + b_ref[...]
    a = jnp.maximum(z, 0.0)
    o_ref[...] = jnp.dot(a, w_ref[...], preferred_element_type=jnp.float32) * dv_ref[...]


def _tc_mid(p0, p1, u, dv, b, W):
    m = u.shape[1]
    k = W.shape[1]
    return pl.pallas_call(
        _mid_kernel,
        grid=(N // BLK,),
        in_specs=[
            pl.BlockSpec((BLK, m), lambda i: (i, 0)),
            pl.BlockSpec((BLK, m), lambda i: (i, 0)),
            pl.BlockSpec((BLK, m), lambda i: (i, 0)),
            pl.BlockSpec((BLK, 1), lambda i: (i, 0)),
            pl.BlockSpec((1, m), lambda i: (0, 0)),
            pl.BlockSpec((m, k), lambda i: (0, 0)),
        ],
        out_specs=pl.BlockSpec((BLK, k), lambda i: (i, 0)),
        out_shape=jax.ShapeDtypeStruct((N, k), jnp.float32),
    )(p0, p1, u, dv, b, W)


def _last_kernel(p0_ref, p1_ref, u_ref, dv_ref, b_ref, o_ref):
    o_ref[...] = (p0_ref[...] + p1_ref[...] + u_ref[...]) * dv_ref[...] + b_ref[...]


def _tc_last(p0, p1, u, dv, b):
    m = u.shape[1]
    return pl.pallas_call(
        _last_kernel,
        grid=(N // BLK,),
        in_specs=[
            pl.BlockSpec((BLK, m), lambda i: (i, 0)),
            pl.BlockSpec((BLK, m), lambda i: (i, 0)),
            pl.BlockSpec((BLK, m), lambda i: (i, 0)),
            pl.BlockSpec((BLK, 1), lambda i: (i, 0)),
            pl.BlockSpec((1, m), lambda i: (0, 0)),
        ],
        out_specs=pl.BlockSpec((BLK, m), lambda i: (i, 0)),
        out_shape=jax.ShapeDtypeStruct((N, m), jnp.float32),
    )(p0, p1, u, dv, b)


def _pad_edges(edge_index):
    per = E // NW
    padn = EP_TILE - per
    row = edge_index[0].reshape(NW, per)
    col = edge_index[1].reshape(NW, per)
    prow = jnp.broadcast_to(jnp.arange(padn, dtype=jnp.int32) % CHUNK, (NW, padn))
    pcol = jnp.broadcast_to(
        N + (jnp.arange(padn, dtype=jnp.int32) % (NPAD - N)), (NW, padn))
    rowp = jnp.concatenate([row, prow], axis=1).reshape(NW, NCHUNK, CHUNK)
    colp = jnp.concatenate([col, pcol], axis=1).reshape(NW, NCHUNK, CHUNK)
    return rowp, colp


def _branch(x, rowi, coli, dv, W1, b1, W2, b2, W3, b3, z128):
    W3p = jnp.pad(W3, ((0, 0), (0, HID - OUT)))
    b3p = jnp.pad(b3, (0, HID - OUT))
    u1 = _tc_first(x, W1, dv)
    S1 = _sc_scatter(u1, rowi, coli, z128, HID)
    u2 = _tc_mid(S1[0, :N], S1[1, :N], u1, dv, b1.reshape(1, HID), W2)
    S2 = _sc_scatter(u2, rowi, coli, z128, HID)
    u3 = _tc_mid(S2[0, :N], S2[1, :N], u2, dv, b2.reshape(1, HID), W3p)
    S3 = _sc_scatter(u3, rowi, coli, z128, HID)
    out = _tc_last(S3[0, :N], S3[1, :N], u3, dv, b3p.reshape(1, HID))
    return out[:, :OUT]


def kernel(x_data_matrix, y_data_matrix, x_edge_index, y_edge_index,
           W1x, b1x, W2x, b2x, W3x, b3x,
           W1y, b1y, W2y, b2y, W3y, b3y):
    rx, cx = _pad_edges(x_edge_index)
    ry, cy = _pad_edges(y_edge_index)
    ones = jnp.ones((CHUNK,), jnp.float32)
    z2n = jnp.zeros((2 * NPAD,), jnp.float32)
    z128 = jnp.zeros((NPAD, HID), jnp.float32)

    degs = _sc_degrees(cx, cy + NPAD, ones, z2n).reshape(NC, 2 * NPAD)
    dinv = _tc_dinv(degs)[0]
    dvx = dinv[:N].reshape(N, 1)
    dvy = dinv[NPAD:NPAD + N].reshape(N, 1)

    xo = _branch(x_data_matrix, rx, cx, dvx, W1x, b1x, W2x, b2x, W3x, b3x, z128)
    yo = _branch(y_data_matrix, ry, cy, dvy, W1y, b1y, W2y, b2y, W3y, b3y, z128)
    return (xo, yo)


# speedup vs baseline: 3.9349x; 1.4366x over previous
import functools

import jax
import jax.numpy as jnp
from jax import lax
from jax.experimental import pallas as pl
from jax.experimental.pallas import tpu as pltpu
from jax.experimental.pallas import tpu_sc as plsc

N = 10000
E = 320000
FIN = 128
HID = 128
OUT = 64

NC = 2
NS = 16
NW = NC * NS
CHUNK = 128
EDGES_PER_TILE = E // NW
NCHUNK = 80
EP_TILE = NCHUNK * CHUNK
EP = EP_TILE * NW
NPAD = 10112
ROWS_PER_TILE = NPAD // NS


def _sc_mesh():
    return plsc.VectorSubcoreMesh(
        core_axis_name="c", subcore_axis_name="s", num_cores=NC, num_subcores=NS)


def _deg_body(cx_hbm, cy_hbm, ones_hbm, zeros_hbm, out_hbm, idx_v, ones_v, acc, sem):
    c = lax.axis_index("c")
    s = lax.axis_index("s")
    wid = c * NS + s

    @pl.when(s == 0)
    def _zero():
        pltpu.sync_copy(zeros_hbm, acc)

    pltpu.sync_copy(ones_hbm, ones_v)
    plsc.subcore_barrier()
    for col_hbm in (cx_hbm, cy_hbm):
        pltpu.sync_copy(col_hbm.at[wid], idx_v)

        def body(j, carry):
            pltpu.sync_copy(ones_v, acc.at[idx_v.at[j]], add=True)
            return carry

        lax.fori_loop(0, NCHUNK, body, 0)
    plsc.subcore_barrier()

    @pl.when(s == 0)
    def _out():
        pltpu.sync_copy(acc, out_hbm.at[c, 0])


def _sc_degrees(cx, cy, ones, zeros):
    return pl.kernel(
        _deg_body,
        out_type=jax.ShapeDtypeStruct((NC, 1, 2 * NPAD), jnp.float32),
        mesh=_sc_mesh(),
        scratch_types=[
            pltpu.VMEM((NCHUNK, CHUNK), jnp.int32),
            pltpu.VMEM((CHUNK,), jnp.float32),
            pltpu.VMEM_SHARED((2 * NPAD,), jnp.float32),
            pltpu.SemaphoreType.DMA,
        ],
    )(cx, cy, ones, zeros)


GCH = 16
NGRP = NCHUNK // GCH


def _scatter_body(d, u_hbm, row_hbm, col_hbm, zeros_hbm, out_hbm,
                  rslot, cidx, buf, acc, gsem0, gsem1, rsem):
    c = lax.axis_index("c")
    s = lax.axis_index("s")
    wid = c * NS + s
    pltpu.sync_copy(zeros_hbm.at[pl.ds(s * ROWS_PER_TILE, ROWS_PER_TILE)],
                    acc.at[pl.ds(s * ROWS_PER_TILE, ROWS_PER_TILE)])
    pltpu.sync_copy(col_hbm.at[wid], cidx)
    pltpu.sync_copy(row_hbm.at[wid, pl.ds(0, GCH)], rslot.at[0])
    plsc.subcore_barrier()

    gsems = (gsem0, gsem1)

    def grp(g, carry):
        @pl.when(g > 0)
        def _wait_idx():
            pltpu.make_async_copy(row_hbm.at[wid, pl.ds(0, GCH)],
                                  rslot.at[0], rsem).wait()

        @pl.when(g < NGRP - 1)
        def _fetch_idx():
            pltpu.async_copy(row_hbm.at[wid, pl.ds((g + 1) * GCH, GCH)],
                             rslot.at[(g + 1) % 2], rsem)

        sg = g % 2
        pltpu.async_copy(u_hbm.at[rslot.at[sg, 0]], buf.at[0], gsem0)
        pltpu.async_copy(u_hbm.at[rslot.at[sg, 1]], buf.at[1], gsem1)

        def pair(p, c2):
            for b in (0, 1):
                k = 2 * p + b
                jj = g * GCH + k
                pltpu.make_async_copy(u_hbm.at[rslot.at[0, 0]],
                                      buf.at[b], gsems[b]).wait()
                pltpu.sync_copy(buf.at[b], acc.at[cidx.at[jj]], add=True)

                @pl.when(k < GCH - 2)
                def _nx():
                    pltpu.async_copy(u_hbm.at[rslot.at[sg, k + 2]],
                                     buf.at[b], gsems[b])

            return c2

        lax.fori_loop(0, GCH // 2, pair, 0)
        return carry

    lax.fori_loop(0, NGRP, grp, 0)
    plsc.subcore_barrier()
    pltpu.sync_copy(acc.at[pl.ds(s * ROWS_PER_TILE, ROWS_PER_TILE)],
                    out_hbm.at[c, pl.ds(s * ROWS_PER_TILE, ROWS_PER_TILE)])


def _sc_scatter(u, rowi, coli, zeros, d):
    return pl.kernel(
        functools.partial(_scatter_body, d),
        out_type=jax.ShapeDtypeStruct((NC, NPAD, d), jnp.float32),
        mesh=_sc_mesh(),
        scratch_types=[
            pltpu.VMEM((2, GCH, CHUNK), jnp.int32),
            pltpu.VMEM((NCHUNK, CHUNK), jnp.int32),
            pltpu.VMEM((2, CHUNK, d), jnp.float32),
            pltpu.VMEM_SHARED((NPAD, d), jnp.float32),
            pltpu.SemaphoreType.DMA,
            pltpu.SemaphoreType.DMA,
            pltpu.SemaphoreType.DMA,
        ],
    )(u, rowi, coli, zeros)


BLK = 1000


def _dinv_kernel(deg_ref, o_ref):
    d = deg_ref[0, :] + deg_ref[1, :] + 1.0
    o_ref[0, :] = lax.rsqrt(d)


def _tc_dinv(degs):
    return pl.pallas_call(
        _dinv_kernel,
        out_shape=jax.ShapeDtypeStruct((1, 2 * NPAD), jnp.float32),
    )(degs)


def _first_kernel(x_ref, w_ref, dv_ref, o_ref):
    h = jnp.dot(x_ref[...], w_ref[...], preferred_element_type=jnp.float32)
    o_ref[...] = h * dv_ref[...]


def _tc_first(x, W, dv):
    m = x.shape[1]
    k = W.shape[1]
    return pl.pallas_call(
        _first_kernel,
        grid=(N // BLK,),
        in_specs=[
            pl.BlockSpec((BLK, m), lambda i: (i, 0)),
            pl.BlockSpec((m, k), lambda i: (0, 0)),
            pl.BlockSpec((BLK, 1), lambda i: (i, 0)),
        ],
        out_specs=pl.BlockSpec((BLK, k), lambda i: (i, 0)),
        out_shape=jax.ShapeDtypeStruct((N, k), jnp.float32),
    )(x, W, dv)


def _mid_kernel(p0_ref, p1_ref, u_ref, dv_ref, b_ref, w_ref, o_ref):
    z = (p0_ref[...] + p1_ref[...] + u_ref[...]) * dv_ref[...] + b_ref[...]
    a = jnp.maximum(z, 0.0)
    o_ref[...] = jnp.dot(a, w_ref[...], preferred_element_type=jnp.float32) * dv_ref[...]


def _tc_mid(p0, p1, u, dv, b, W):
    m = u.shape[1]
    k = W.shape[1]
    return pl.pallas_call(
        _mid_kernel,
        grid=(N // BLK,),
        in_specs=[
            pl.BlockSpec((BLK, m), lambda i: (i, 0)),
            pl.BlockSpec((BLK, m), lambda i: (i, 0)),
            pl.BlockSpec((BLK, m), lambda i: (i, 0)),
            pl.BlockSpec((BLK, 1), lambda i: (i, 0)),
            pl.BlockSpec((1, m), lambda i: (0, 0)),
            pl.BlockSpec((m, k), lambda i: (0, 0)),
        ],
        out_specs=pl.BlockSpec((BLK, k), lambda i: (i, 0)),
        out_shape=jax.ShapeDtypeStruct((N, k), jnp.float32),
    )(p0, p1, u, dv, b, W)


def _last_kernel(p0_ref, p1_ref, u_ref, dv_ref, b_ref, o_ref):
    o_ref[...] = (p0_ref[...] + p1_ref[...] + u_ref[...]) * dv_ref[...] + b_ref[...]


def _tc_last(p0, p1, u, dv, b):
    m = u.shape[1]
    return pl.pallas_call(
        _last_kernel,
        grid=(N // BLK,),
        in_specs=[
            pl.BlockSpec((BLK, m), lambda i: (i, 0)),
            pl.BlockSpec((BLK, m), lambda i: (i, 0)),
            pl.BlockSpec((BLK, m), lambda i: (i, 0)),
            pl.BlockSpec((BLK, 1), lambda i: (i, 0)),
            pl.BlockSpec((1, m), lambda i: (0, 0)),
        ],
        out_specs=pl.BlockSpec((BLK, m), lambda i: (i, 0)),
        out_shape=jax.ShapeDtypeStruct((N, m), jnp.float32),
    )(p0, p1, u, dv, b)


def _pad_edges(edge_index):
    per = E // NW
    padn = EP_TILE - per
    row = edge_index[0].reshape(NW, per)
    col = edge_index[1].reshape(NW, per)
    prow = jnp.broadcast_to(jnp.arange(padn, dtype=jnp.int32) % CHUNK, (NW, padn))
    pcol = jnp.broadcast_to(
        N + (jnp.arange(padn, dtype=jnp.int32) % (NPAD - N)), (NW, padn))
    rowp = jnp.concatenate([row, prow], axis=1).reshape(NW, NCHUNK, CHUNK)
    colp = jnp.concatenate([col, pcol], axis=1).reshape(NW, NCHUNK, CHUNK)
    return rowp, colp


def _branch(x, rowi, coli, dv, W1, b1, W2, b2, W3, b3, z128):
    W3p = jnp.pad(W3, ((0, 0), (0, HID - OUT)))
    b3p = jnp.pad(b3, (0, HID - OUT))
    u1 = _tc_first(x, W1, dv)
    S1 = _sc_scatter(u1, rowi, coli, z128, HID)
    u2 = _tc_mid(S1[0, :N], S1[1, :N], u1, dv, b1.reshape(1, HID), W2)
    S2 = _sc_scatter(u2, rowi, coli, z128, HID)
    u3 = _tc_mid(S2[0, :N], S2[1, :N], u2, dv, b2.reshape(1, HID), W3p)
    S3 = _sc_scatter(u3, rowi, coli, z128, HID)
    out = _tc_last(S3[0, :N], S3[1, :N], u3, dv, b3p.reshape(1, HID))
    return out[:, :OUT]


def kernel(x_data_matrix, y_data_matrix, x_edge_index, y_edge_index,
           W1x, b1x, W2x, b2x, W3x, b3x,
           W1y, b1y, W2y, b2y, W3y, b3y):
    rx, cx = _pad_edges(x_edge_index)
    ry, cy = _pad_edges(y_edge_index)
    ones = jnp.ones((CHUNK,), jnp.float32)
    z2n = jnp.zeros((2 * NPAD,), jnp.float32)
    z128 = jnp.zeros((NPAD, HID), jnp.float32)

    degs = _sc_degrees(cx, cy + NPAD, ones, z2n).reshape(NC, 2 * NPAD)
    dinv = _tc_dinv(degs)[0]
    dvx = dinv[:N].reshape(N, 1)
    dvy = dinv[NPAD:NPAD + N].reshape(N, 1)

    xo = _branch(x_data_matrix, rx, cx, dvx, W1x, b1x, W2x, b2x, W3x, b3x, z128)
    yo = _branch(y_data_matrix, ry, cy, dvy, W1y, b1y, W2y, b2y, W3y, b3y, z128)
    return (xo, yo)


# speedup vs baseline: 3.9787x; 1.0111x over previous
import functools

import jax
import jax.numpy as jnp
from jax import lax
from jax.experimental import pallas as pl
from jax.experimental.pallas import tpu as pltpu
from jax.experimental.pallas import tpu_sc as plsc

N = 10000
E = 320000
FIN = 128
HID = 128
OUT = 64

NC = 2
NS = 16
NW = NC * NS
CHUNK = 128
EDGES_PER_TILE = E // NW
NCHUNK = 80
EP_TILE = NCHUNK * CHUNK
EP = EP_TILE * NW
NPAD = 10112
ROWS_PER_TILE = NPAD // NS


def _sc_mesh():
    return plsc.VectorSubcoreMesh(
        core_axis_name="c", subcore_axis_name="s", num_cores=NC, num_subcores=NS)


def _deg_body(cx_hbm, cy_hbm, ones_hbm, zeros_hbm, out_hbm, idx_v, ones_v, acc, sem):
    c = lax.axis_index("c")
    s = lax.axis_index("s")
    wid = c * NS + s

    @pl.when(s == 0)
    def _zero():
        pltpu.sync_copy(zeros_hbm, acc)

    pltpu.sync_copy(ones_hbm, ones_v)
    plsc.subcore_barrier()
    for col_hbm in (cx_hbm, cy_hbm):
        pltpu.sync_copy(col_hbm.at[wid], idx_v)

        def fire(j, carry):
            pltpu.async_copy(ones_v, acc.at[idx_v.at[j]], sem, add=True)
            return carry

        lax.fori_loop(0, NCHUNK, fire, 0)

        def drain(j, carry):
            pltpu.make_async_copy(ones_v, acc.at[idx_v.at[0]], sem).wait()
            return carry

        lax.fori_loop(0, NCHUNK, drain, 0)
    plsc.subcore_barrier()

    @pl.when(s == 0)
    def _out():
        pltpu.sync_copy(acc, out_hbm.at[c, 0])


def _sc_degrees(cx, cy, ones, zeros):
    return pl.kernel(
        _deg_body,
        out_type=jax.ShapeDtypeStruct((NC, 1, 2 * NPAD), jnp.float32),
        mesh=_sc_mesh(),
        scratch_types=[
            pltpu.VMEM((NCHUNK, CHUNK), jnp.int32),
            pltpu.VMEM((CHUNK,), jnp.float32),
            pltpu.VMEM_SHARED((2 * NPAD,), jnp.float32),
            pltpu.SemaphoreType.DMA,
        ],
    )(cx, cy, ones, zeros)


GCH = 16
NGRP = NCHUNK // GCH


def _scatter_body(d, u_hbm, row_hbm, col_hbm, zeros_hbm, out_hbm,
                  rslot, cidx, buf, acc, gsem0, gsem1, rsem):
    c = lax.axis_index("c")
    s = lax.axis_index("s")
    wid = c * NS + s
    pltpu.sync_copy(zeros_hbm.at[pl.ds(s * ROWS_PER_TILE, ROWS_PER_TILE)],
                    acc.at[pl.ds(s * ROWS_PER_TILE, ROWS_PER_TILE)])
    pltpu.sync_copy(col_hbm.at[wid], cidx)
    pltpu.sync_copy(row_hbm.at[wid, pl.ds(0, GCH)], rslot.at[0])
    plsc.subcore_barrier()

    gsems = (gsem0, gsem1)

    def grp(g, carry):
        @pl.when(g > 0)
        def _wait_idx():
            pltpu.make_async_copy(row_hbm.at[wid, pl.ds(0, GCH)],
                                  rslot.at[0], rsem).wait()

        @pl.when(g < NGRP - 1)
        def _fetch_idx():
            pltpu.async_copy(row_hbm.at[wid, pl.ds((g + 1) * GCH, GCH)],
                             rslot.at[(g + 1) % 2], rsem)

        sg = g % 2
        pltpu.async_copy(u_hbm.at[rslot.at[sg, 0]], buf.at[0], gsem0)
        pltpu.async_copy(u_hbm.at[rslot.at[sg, 1]], buf.at[1], gsem1)

        def pair(p, c2):
            for b in (0, 1):
                k = 2 * p + b
                jj = g * GCH + k
                pltpu.make_async_copy(u_hbm.at[rslot.at[0, 0]],
                                      buf.at[b], gsems[b]).wait()
                pltpu.sync_copy(buf.at[b], acc.at[cidx.at[jj]], add=True)

                @pl.when(k < GCH - 2)
                def _nx():
                    pltpu.async_copy(u_hbm.at[rslot.at[sg, k + 2]],
                                     buf.at[b], gsems[b])

            return c2

        lax.fori_loop(0, GCH // 2, pair, 0)
        return carry

    lax.fori_loop(0, NGRP, grp, 0)
    plsc.subcore_barrier()
    pltpu.sync_copy(acc.at[pl.ds(s * ROWS_PER_TILE, ROWS_PER_TILE)],
                    out_hbm.at[c, pl.ds(s * ROWS_PER_TILE, ROWS_PER_TILE)])


def _sc_scatter(u, rowi, coli, zeros, d):
    return pl.kernel(
        functools.partial(_scatter_body, d),
        out_type=jax.ShapeDtypeStruct((NC, NPAD, d), jnp.float32),
        mesh=_sc_mesh(),
        scratch_types=[
            pltpu.VMEM((2, GCH, CHUNK), jnp.int32),
            pltpu.VMEM((NCHUNK, CHUNK), jnp.int32),
            pltpu.VMEM((2, CHUNK, d), jnp.float32),
            pltpu.VMEM_SHARED((NPAD, d), jnp.float32),
            pltpu.SemaphoreType.DMA,
            pltpu.SemaphoreType.DMA,
            pltpu.SemaphoreType.DMA,
        ],
    )(u, rowi, coli, zeros)


BLK = 1000


def _dinv_kernel(deg_ref, o_ref):
    d = deg_ref[0, :] + deg_ref[1, :] + 1.0
    o_ref[0, :] = lax.rsqrt(d)


def _tc_dinv(degs):
    return pl.pallas_call(
        _dinv_kernel,
        out_shape=jax.ShapeDtypeStruct((1, 2 * NPAD), jnp.float32),
    )(degs)


def _first_kernel(x_ref, w_ref, dv_ref, o_ref):
    h = jnp.dot(x_ref[...], w_ref[...], preferred_element_type=jnp.float32)
    o_ref[...] = h * dv_ref[...]


def _tc_first(x, W, dv):
    m = x.shape[1]
    k = W.shape[1]
    return pl.pallas_call(
        _first_kernel,
        grid=(N // BLK,),
        in_specs=[
            pl.BlockSpec((BLK, m), lambda i: (i, 0)),
            pl.BlockSpec((m, k), lambda i: (0, 0)),
            pl.BlockSpec((BLK, 1), lambda i: (i, 0)),
        ],
        out_specs=pl.BlockSpec((BLK, k), lambda i: (i, 0)),
        out_shape=jax.ShapeDtypeStruct((N, k), jnp.float32),
    )(x, W, dv)


def _mid_kernel(p0_ref, p1_ref, u_ref, dv_ref, b_ref, w_ref, o_ref):
    z = (p0_ref[...] + p1_ref[...] + u_ref[...]) * dv_ref[...] + b_ref[...]
    a = jnp.maximum(z, 0.0)
    o_ref[...] = jnp.dot(a, w_ref[...], preferred_element_type=jnp.float32) * dv_ref[...]


def _tc_mid(p0, p1, u, dv, b, W):
    m = u.shape[1]
    k = W.shape[1]
    return pl.pallas_call(
        _mid_kernel,
        grid=(N // BLK,),
        in_specs=[
            pl.BlockSpec((BLK, m), lambda i: (i, 0)),
            pl.BlockSpec((BLK, m), lambda i: (i, 0)),
            pl.BlockSpec((BLK, m), lambda i: (i, 0)),
            pl.BlockSpec((BLK, 1), lambda i: (i, 0)),
            pl.BlockSpec((1, m), lambda i: (0, 0)),
            pl.BlockSpec((m, k), lambda i: (0, 0)),
        ],
        out_specs=pl.BlockSpec((BLK, k), lambda i: (i, 0)),
        out_shape=jax.ShapeDtypeStruct((N, k), jnp.float32),
    )(p0, p1, u, dv, b, W)


def _last_kernel(p0_ref, p1_ref, u_ref, dv_ref, b_ref, o_ref):
    o_ref[...] = (p0_ref[...] + p1_ref[...] + u_ref[...]) * dv_ref[...] + b_ref[...]


def _tc_last(p0, p1, u, dv, b):
    m = u.shape[1]
    return pl.pallas_call(
        _last_kernel,
        grid=(N // BLK,),
        in_specs=[
            pl.BlockSpec((BLK, m), lambda i: (i, 0)),
            pl.BlockSpec((BLK, m), lambda i: (i, 0)),
            pl.BlockSpec((BLK, m), lambda i: (i, 0)),
            pl.BlockSpec((BLK, 1), lambda i: (i, 0)),
            pl.BlockSpec((1, m), lambda i: (0, 0)),
        ],
        out_specs=pl.BlockSpec((BLK, m), lambda i: (i, 0)),
        out_shape=jax.ShapeDtypeStruct((N, m), jnp.float32),
    )(p0, p1, u, dv, b)


def _pad_edges(edge_index):
    per = E // NW
    padn = EP_TILE - per
    row = edge_index[0].reshape(NW, per)
    col = edge_index[1].reshape(NW, per)
    prow = jnp.broadcast_to(jnp.arange(padn, dtype=jnp.int32) % CHUNK, (NW, padn))
    pcol = jnp.broadcast_to(
        N + (jnp.arange(padn, dtype=jnp.int32) % (NPAD - N)), (NW, padn))
    rowp = jnp.concatenate([row, prow], axis=1).reshape(NW, NCHUNK, CHUNK)
    colp = jnp.concatenate([col, pcol], axis=1).reshape(NW, NCHUNK, CHUNK)
    return rowp, colp


def _branch(x, rowi, coli, dv, W1, b1, W2, b2, W3, b3, z128):
    W3p = jnp.pad(W3, ((0, 0), (0, HID - OUT)))
    b3p = jnp.pad(b3, (0, HID - OUT))
    u1 = _tc_first(x, W1, dv)
    S1 = _sc_scatter(u1, rowi, coli, z128, HID)
    u2 = _tc_mid(S1[0, :N], S1[1, :N], u1, dv, b1.reshape(1, HID), W2)
    S2 = _sc_scatter(u2, rowi, coli, z128, HID)
    u3 = _tc_mid(S2[0, :N], S2[1, :N], u2, dv, b2.reshape(1, HID), W3p)
    S3 = _sc_scatter(u3, rowi, coli, z128, HID)
    out = _tc_last(S3[0, :N], S3[1, :N], u3, dv, b3p.reshape(1, HID))
    return out[:, :OUT]


def kernel(x_data_matrix, y_data_matrix, x_edge_index, y_edge_index,
           W1x, b1x, W2x, b2x, W3x, b3x,
           W1y, b1y, W2y, b2y, W3y, b3y):
    rx, cx = _pad_edges(x_edge_index)
    ry, cy = _pad_edges(y_edge_index)
    ones = jnp.ones((CHUNK,), jnp.float32)
    z2n = jnp.zeros((2 * NPAD,), jnp.float32)
    z128 = jnp.zeros((NPAD, HID), jnp.float32)

    degs = _sc_degrees(cx, cy + NPAD, ones, z2n).reshape(NC, 2 * NPAD)
    dinv = _tc_dinv(degs)[0]
    dvx = dinv[:N].reshape(N, 1)
    dvy = dinv[NPAD:NPAD + N].reshape(N, 1)

    xo = _branch(x_data_matrix, rx, cx, dvx, W1x, b1x, W2x, b2x, W3x, b3x, z128)
    yo = _branch(y_data_matrix, ry, cy, dvy, W1y, b1y, W2y, b2y, W3y, b3y, z128)
    return (xo, yo)


# speedup vs baseline: 3.9820x; 1.0008x over previous
import functools

import jax
import jax.numpy as jnp
from jax import lax
from jax.experimental import pallas as pl
from jax.experimental.pallas import tpu as pltpu
from jax.experimental.pallas import tpu_sc as plsc

N = 10000
E = 320000
FIN = 128
HID = 128
OUT = 64

NC = 2
NS = 16
NW = NC * NS
CHUNK = 128
EDGES_PER_TILE = E // NW
NCHUNK = 80
EP_TILE = NCHUNK * CHUNK
EP = EP_TILE * NW
NPAD = 10112
ROWS_PER_TILE = NPAD // NS


def _sc_mesh():
    return plsc.VectorSubcoreMesh(
        core_axis_name="c", subcore_axis_name="s", num_cores=NC, num_subcores=NS)


def _deg_body(cx_hbm, cy_hbm, ones_hbm, zeros_hbm, out_hbm, idx_v, ones_v, acc, sem):
    c = lax.axis_index("c")
    s = lax.axis_index("s")
    wid = c * NS + s

    @pl.when(s == 0)
    def _zero():
        pltpu.sync_copy(zeros_hbm, acc)

    pltpu.sync_copy(ones_hbm, ones_v)
    plsc.subcore_barrier()
    for col_hbm in (cx_hbm, cy_hbm):
        pltpu.sync_copy(col_hbm.at[wid], idx_v)

        def fire(j, carry):
            pltpu.async_copy(ones_v, acc.at[idx_v.at[j]], sem, add=True)
            return carry

        lax.fori_loop(0, NCHUNK, fire, 0)

        def drain(j, carry):
            pltpu.make_async_copy(ones_v, acc.at[idx_v.at[0]], sem).wait()
            return carry

        lax.fori_loop(0, NCHUNK, drain, 0)
    plsc.subcore_barrier()

    @pl.when(s == 0)
    def _out():
        pltpu.sync_copy(acc, out_hbm.at[c, 0])


def _sc_degrees(cx, cy, ones, zeros):
    return pl.kernel(
        _deg_body,
        out_type=jax.ShapeDtypeStruct((NC, 1, 2 * NPAD), jnp.float32),
        mesh=_sc_mesh(),
        scratch_types=[
            pltpu.VMEM((NCHUNK, CHUNK), jnp.int32),
            pltpu.VMEM((CHUNK,), jnp.float32),
            pltpu.VMEM_SHARED((2 * NPAD,), jnp.float32),
            pltpu.SemaphoreType.DMA,
        ],
    )(cx, cy, ones, zeros)


GCH = 16
NGRP = NCHUNK // GCH


def _scatter_body(d, u_hbm, row_hbm, col_hbm, zeros_hbm, out_hbm,
                  rslot, cidx, buf, acc, gsem0, gsem1, rsem):
    c = lax.axis_index("c")
    s = lax.axis_index("s")
    wid = c * NS + s
    pltpu.sync_copy(zeros_hbm.at[pl.ds(s * ROWS_PER_TILE, ROWS_PER_TILE)],
                    acc.at[pl.ds(s * ROWS_PER_TILE, ROWS_PER_TILE)])
    pltpu.sync_copy(col_hbm.at[wid], cidx)
    pltpu.sync_copy(row_hbm.at[wid, pl.ds(0, GCH)], rslot.at[0])
    plsc.subcore_barrier()

    gsems = (gsem0, gsem1)

    def grp(g, carry):
        @pl.when(g > 0)
        def _wait_idx():
            pltpu.make_async_copy(row_hbm.at[wid, pl.ds(0, GCH)],
                                  rslot.at[0], rsem).wait()

        @pl.when(g < NGRP - 1)
        def _fetch_idx():
            pltpu.async_copy(row_hbm.at[wid, pl.ds((g + 1) * GCH, GCH)],
                             rslot.at[(g + 1) % 2], rsem)

        sg = g % 2
        pltpu.async_copy(u_hbm.at[rslot.at[sg, 0]], buf.at[0], gsem0)
        pltpu.async_copy(u_hbm.at[rslot.at[sg, 1]], buf.at[1], gsem1)

        def pair(p, c2):
            for b in (0, 1):
                k = 2 * p + b
                jj = g * GCH + k
                pltpu.make_async_copy(u_hbm.at[rslot.at[0, 0]],
                                      buf.at[b], gsems[b]).wait()
                pltpu.sync_copy(buf.at[b], acc.at[cidx.at[jj]], add=True)

                @pl.when(k < GCH - 2)
                def _nx():
                    pltpu.async_copy(u_hbm.at[rslot.at[sg, k + 2]],
                                     buf.at[b], gsems[b])

            return c2

        lax.fori_loop(0, GCH // 2, pair, 0)
        return carry

    lax.fori_loop(0, NGRP, grp, 0)
    plsc.subcore_barrier()
    pltpu.sync_copy(acc.at[pl.ds(s * ROWS_PER_TILE, ROWS_PER_TILE)],
                    out_hbm.at[c, pl.ds(s * ROWS_PER_TILE, ROWS_PER_TILE)])


def _sc_scatter(u, rowi, coli, zeros, d):
    return pl.kernel(
        functools.partial(_scatter_body, d),
        out_type=jax.ShapeDtypeStruct((NC, NPAD, d), jnp.float32),
        mesh=_sc_mesh(),
        scratch_types=[
            pltpu.VMEM((2, GCH, CHUNK), jnp.int32),
            pltpu.VMEM((NCHUNK, CHUNK), jnp.int32),
            pltpu.VMEM((2, CHUNK, d), jnp.float32),
            pltpu.VMEM_SHARED((NPAD, d), jnp.float32),
            pltpu.SemaphoreType.DMA,
            pltpu.SemaphoreType.DMA,
            pltpu.SemaphoreType.DMA,
        ],
    )(u, rowi, coli, zeros)


BLK = 1000


def _dinv_kernel(deg_ref, o_ref):
    d = deg_ref[0, :] + deg_ref[1, :] + 1.0
    o_ref[0, :] = lax.rsqrt(d)


def _tc_dinv(degs):
    return pl.pallas_call(
        _dinv_kernel,
        out_shape=jax.ShapeDtypeStruct((1, 2 * NPAD), jnp.float32),
    )(degs)


def _first_kernel(x_ref, w_ref, dv_ref, o_ref):
    h = jnp.dot(x_ref[...], w_ref[...], preferred_element_type=jnp.float32)
    o_ref[...] = h * dv_ref[...]


def _tc_first(x, W, dv):
    m = x.shape[1]
    k = W.shape[1]
    return pl.pallas_call(
        _first_kernel,
        grid=(N // BLK,),
        in_specs=[
            pl.BlockSpec((BLK, m), lambda i: (i, 0)),
            pl.BlockSpec((m, k), lambda i: (0, 0)),
            pl.BlockSpec((BLK, 1), lambda i: (i, 0)),
        ],
        out_specs=pl.BlockSpec((BLK, k), lambda i: (i, 0)),
        out_shape=jax.ShapeDtypeStruct((N, k), jnp.float32),
    )(x, W, dv)


def _mid_kernel(p0_ref, p1_ref, u_ref, dv_ref, b_ref, w_ref, o_ref):
    z = (p0_ref[...] + p1_ref[...] + u_ref[...]) * dv_ref[...] + b_ref[...]
    a = jnp.maximum(z, 0.0)
    o_ref[...] = jnp.dot(a, w_ref[...], preferred_element_type=jnp.float32) * dv_ref[...]


def _tc_mid(p0, p1, u, dv, b, W):
    m = u.shape[1]
    k = W.shape[1]
    return pl.pallas_call(
        _mid_kernel,
        grid=(N // BLK,),
        in_specs=[
            pl.BlockSpec((BLK, m), lambda i: (i, 0)),
            pl.BlockSpec((BLK, m), lambda i: (i, 0)),
            pl.BlockSpec((BLK, m), lambda i: (i, 0)),
            pl.BlockSpec((BLK, 1), lambda i: (i, 0)),
            pl.BlockSpec((1, m), lambda i: (0, 0)),
            pl.BlockSpec((m, k), lambda i: (0, 0)),
        ],
        out_specs=pl.BlockSpec((BLK, k), lambda i: (i, 0)),
        out_shape=jax.ShapeDtypeStruct((N, k), jnp.float32),
    )(p0, p1, u, dv, b, W)


def _last_kernel(p0_ref, p1_ref, u_ref, dv_ref, b_ref, o_ref):
    o_ref[...] = (p0_ref[...] + p1_ref[...] + u_ref[...]) * dv_ref[...] + b_ref[...]


def _tc_last(p0, p1, u, dv, b):
    m = u.shape[1]
    return pl.pallas_call(
        _last_kernel,
        grid=(N // BLK,),
        in_specs=[
            pl.BlockSpec((BLK, m), lambda i: (i, 0)),
            pl.BlockSpec((BLK, m), lambda i: (i, 0)),
            pl.BlockSpec((BLK, m), lambda i: (i, 0)),
            pl.BlockSpec((BLK, 1), lambda i: (i, 0)),
            pl.BlockSpec((1, m), lambda i: (0, 0)),
        ],
        out_specs=pl.BlockSpec((BLK, m), lambda i: (i, 0)),
        out_shape=jax.ShapeDtypeStruct((N, m), jnp.float32),
    )(p0, p1, u, dv, b)


def _pad_edges(edge_index):
    per = E // NW
    padn = EP_TILE - per
    row = edge_index[0].reshape(NW, per)
    col = edge_index[1].reshape(NW, per)
    prow = jnp.broadcast_to(jnp.arange(padn, dtype=jnp.int32) % CHUNK, (NW, padn))
    pcol = jnp.broadcast_to(
        N + (jnp.arange(padn, dtype=jnp.int32) % (NPAD - N)), (NW, padn))
    rowp = jnp.concatenate([row, prow], axis=1).reshape(NW, NCHUNK, CHUNK)
    colp = jnp.concatenate([col, pcol], axis=1).reshape(NW, NCHUNK, CHUNK)
    return rowp, colp


def _two_branches(x, rx, cx, dvx, Wx, y, ry, cy, dvy, Wy, z128):
    (W1x, b1x, W2x, b2x, W3x, b3x) = Wx
    (W1y, b1y, W2y, b2y, W3y, b3y) = Wy
    W3xp = jnp.pad(W3x, ((0, 0), (0, HID - OUT)))
    b3xp = jnp.pad(b3x, (0, HID - OUT)).reshape(1, HID)
    W3yp = jnp.pad(W3y, ((0, 0), (0, HID - OUT)))
    b3yp = jnp.pad(b3y, (0, HID - OUT)).reshape(1, HID)

    u1x = _tc_first(x, W1x, dvx)
    S1x = _sc_scatter(u1x, rx, cx, z128, HID)
    u1y = _tc_first(y, W1y, dvy)
    S1y = _sc_scatter(u1y, ry, cy, z128, HID)
    u2x = _tc_mid(S1x[0, :N], S1x[1, :N], u1x, dvx, b1x.reshape(1, HID), W2x)
    S2x = _sc_scatter(u2x, rx, cx, z128, HID)
    u2y = _tc_mid(S1y[0, :N], S1y[1, :N], u1y, dvy, b1y.reshape(1, HID), W2y)
    S2y = _sc_scatter(u2y, ry, cy, z128, HID)
    u3x = _tc_mid(S2x[0, :N], S2x[1, :N], u2x, dvx, b2x.reshape(1, HID), W3xp)
    S3x = _sc_scatter(u3x, rx, cx, z128, HID)
    u3y = _tc_mid(S2y[0, :N], S2y[1, :N], u2y, dvy, b2y.reshape(1, HID), W3yp)
    S3y = _sc_scatter(u3y, ry, cy, z128, HID)
    xo = _tc_last(S3x[0, :N], S3x[1, :N], u3x, dvx, b3xp)
    yo = _tc_last(S3y[0, :N], S3y[1, :N], u3y, dvy, b3yp)
    return xo[:, :OUT], yo[:, :OUT]


def kernel(x_data_matrix, y_data_matrix, x_edge_index, y_edge_index,
           W1x, b1x, W2x, b2x, W3x, b3x,
           W1y, b1y, W2y, b2y, W3y, b3y):
    rx, cx = _pad_edges(x_edge_index)
    ry, cy = _pad_edges(y_edge_index)
    ones = jnp.ones((CHUNK,), jnp.float32)
    z2n = jnp.zeros((2 * NPAD,), jnp.float32)
    z128 = jnp.zeros((NPAD, HID), jnp.float32)

    degs = _sc_degrees(cx, cy + NPAD, ones, z2n).reshape(NC, 2 * NPAD)
    dinv = _tc_dinv(degs)[0]
    dvx = dinv[:N].reshape(N, 1)
    dvy = dinv[NPAD:NPAD + N].reshape(N, 1)

    xo, yo = _two_branches(
        x_data_matrix, rx, cx, dvx, (W1x, b1x, W2x, b2x, W3x, b3x),
        y_data_matrix, ry, cy, dvy, (W1y, b1y, W2y, b2y, W3y, b3y), z128)
    return (xo, yo)
